# baseline (device time: 219461 ns/iter reference)
import jax
import jax.numpy as jnp
from jax import lax
from jax.experimental import pallas as pl
from jax.experimental.pallas import tpu as pltpu

N_DEV = 8
M = 1536
N = 1536
CHUNK = M // N_DEV


def kernel(A, B):
    def body(a_ref, b_ref, out_ref, p_ref, comm_ref, send_sems, recv_sems,
             credit_sem):
        my = lax.axis_index("i")
        left = lax.rem(my + N_DEV - 1, N_DEV)
        right = lax.rem(my + 1, N_DEV)

        barrier = pltpu.get_barrier_semaphore()
        for nbr in (left, right):
            pl.semaphore_signal(barrier, inc=1, device_id=(nbr,),
                                device_id_type=pl.DeviceIdType.MESH)
        pl.semaphore_wait(barrier, 2)

        p_ref[...] = lax.dot_general(
            a_ref[...], b_ref[...], (((1,), (0,)), ((), ())),
            preferred_element_type=jnp.float32)

        def rows(idx):
            return pl.ds(idx * CHUNK, CHUNK)

        comm_ref[0, :, :] = p_ref[rows(lax.rem(my + N_DEV - 1, N_DEV)), :]

        for t in range(2 * (N_DEV - 1)):
            send_slot = t % 2
            recv_slot = (t + 1) % 2
            if t >= 2:
                pl.semaphore_wait(credit_sem, 1)
            rdma = pltpu.make_async_remote_copy(
                src_ref=comm_ref.at[send_slot],
                dst_ref=comm_ref.at[recv_slot],
                send_sem=send_sems.at[send_slot],
                recv_sem=recv_sems.at[recv_slot],
                device_id=(right,),
                device_id_type=pl.DeviceIdType.MESH,
            )
            rdma.start()
            rdma.wait()

            if t < N_DEV - 1:
                c = lax.rem(my + 2 * N_DEV - 2 - t, N_DEV)
                acc = comm_ref[recv_slot, :, :] + p_ref[rows(c), :]
                if t == N_DEV - 2:
                    act = acc / (1.0 + jnp.exp(-acc))
                    out_ref[rows(c), :] = act
                    comm_ref[recv_slot, :, :] = act
                else:
                    comm_ref[recv_slot, :, :] = acc
            else:
                h = t - (N_DEV - 1)
                c = lax.rem(my + 2 * N_DEV - 1 - h, N_DEV)
                out_ref[rows(c), :] = comm_ref[recv_slot, :, :]

            if t < 2 * (N_DEV - 1) - 2:
                pl.semaphore_signal(credit_sem, inc=1, device_id=(left,),
                                    device_id_type=pl.DeviceIdType.MESH)

    return pl.pallas_call(
        body,
        out_shape=jax.ShapeDtypeStruct((M, N), jnp.float32),
        in_specs=[pl.BlockSpec(memory_space=pltpu.VMEM),
                  pl.BlockSpec(memory_space=pltpu.VMEM)],
        out_specs=pl.BlockSpec(memory_space=pltpu.VMEM),
        scratch_shapes=[
            pltpu.VMEM((M, N), jnp.float32),
            pltpu.VMEM((2, CHUNK, N), jnp.float32),
            pltpu.SemaphoreType.DMA((2,)),
            pltpu.SemaphoreType.DMA((2,)),
            pltpu.SemaphoreType.REGULAR,
        ],
        compiler_params=pltpu.CompilerParams(collective_id=0),
    )(A, B)


# device time: 87126 ns/iter; 2.5189x vs baseline; 2.5189x over previous
import jax
import jax.numpy as jnp
from jax import lax
from jax.experimental import pallas as pl
from jax.experimental.pallas import tpu as pltpu

N_DEV = 8
M = 1536
N = 1536
P_PARTS = 3
PART_ROWS = M // P_PARTS
MASK_X, MASK_Y, MASK_Z = 1, 3, 4
def _side_bit(my, mask):
    if mask == MASK_X:
        return (my ^ (my >> 1)) & 1
    if mask == MASK_Y:
        return (my >> 1) & 1
    return (my >> 2) & 1
ORDER = [
    [MASK_X, MASK_Y, MASK_Z],
    [MASK_Y, MASK_Z, MASK_X],
    [MASK_Z, MASK_X, MASK_Y],
]
EX = [PART_ROWS // 2, PART_ROWS // 4, PART_ROWS // 8]
SCR_OFF = [0, EX[0], EX[0] + EX[1]]
SCR_ROWS = EX[0] + EX[1] + EX[2]


def kernel(A, B):
    def body(a_ref, b_ref, out_ref, p_ref, scr_ref,
             rs_send, rs_recv, ag_send, ag_recv):
        my = lax.axis_index("i")

        barrier = pltpu.get_barrier_semaphore()
        for mask in (MASK_X, MASK_Y, MASK_Z):
            pl.semaphore_signal(barrier, inc=1, device_id=(my ^ mask,),
                                device_id_type=pl.DeviceIdType.MESH)
        pl.semaphore_wait(barrier, 3)

        p_ref[...] = lax.dot_general(
            a_ref[...], b_ref[...], (((1,), (0,)), ((), ())),
            preferred_element_type=jnp.float32)

        lo = [0, 0, 0]
        for s in range(3):
            ex = EX[s]
            started = []
            for part in range(P_PARTS):
                mask = ORDER[part][s]
                partner = my ^ mask
                bbit = _side_bit(my, mask)
                send_lo = lo[part] + (1 - bbit) * ex
                keep_lo = lo[part] + bbit * ex
                scr_base = part * SCR_ROWS + SCR_OFF[s]
                rdma = pltpu.make_async_remote_copy(
                    src_ref=p_ref.at[pl.ds(part * PART_ROWS + send_lo, ex), :],
                    dst_ref=scr_ref.at[pl.ds(scr_base, ex), :],
                    send_sem=rs_send.at[part, s],
                    recv_sem=rs_recv.at[part, s],
                    device_id=(partner,),
                    device_id_type=pl.DeviceIdType.MESH,
                )
                rdma.start()
                started.append((rdma, part, keep_lo, scr_base))
                lo[part] = keep_lo
            for rdma, part, keep_lo, scr_base in started:
                rdma.wait()
                g = part * PART_ROWS + keep_lo
                p_ref[pl.ds(g, ex), :] = (
                    p_ref[pl.ds(g, ex), :] + scr_ref[pl.ds(scr_base, ex), :])

        for part in range(P_PARTS):
            g = part * PART_ROWS + lo[part]
            z = p_ref[pl.ds(g, EX[2]), :]
            out_ref[pl.ds(g, EX[2]), :] = z / (1.0 + jnp.exp(-z))

        own = EX[2]
        for k, s in enumerate((2, 1, 0)):
            started = []
            for part in range(P_PARTS):
                mask = ORDER[part][s]
                partner = my ^ mask
                bbit = _side_bit(my, mask)
                g = part * PART_ROWS + lo[part]
                rdma = pltpu.make_async_remote_copy(
                    src_ref=out_ref.at[pl.ds(g, own), :],
                    dst_ref=out_ref.at[pl.ds(g, own), :],
                    send_sem=ag_send.at[part, k],
                    recv_sem=ag_recv.at[part, k],
                    device_id=(partner,),
                    device_id_type=pl.DeviceIdType.MESH,
                )
                rdma.start()
                started.append(rdma)
                lo[part] = lo[part] - bbit * own
            for rdma in started:
                rdma.wait()
            own *= 2

    return pl.pallas_call(
        body,
        out_shape=jax.ShapeDtypeStruct((M, N), jnp.float32),
        in_specs=[pl.BlockSpec(memory_space=pltpu.VMEM),
                  pl.BlockSpec(memory_space=pltpu.VMEM)],
        out_specs=pl.BlockSpec(memory_space=pltpu.VMEM),
        scratch_shapes=[
            pltpu.VMEM((M, N), jnp.float32),
            pltpu.VMEM((P_PARTS * SCR_ROWS, N), jnp.float32),
            pltpu.SemaphoreType.DMA((P_PARTS, 3)),
            pltpu.SemaphoreType.DMA((P_PARTS, 3)),
            pltpu.SemaphoreType.DMA((P_PARTS, 3)),
            pltpu.SemaphoreType.DMA((P_PARTS, 3)),
        ],
        compiler_params=pltpu.CompilerParams(collective_id=0),
    )(A, B)


# device time: 56396 ns/iter; 3.8914x vs baseline; 1.5449x over previous
import jax
import jax.numpy as jnp
from jax import lax
from jax.experimental import pallas as pl
from jax.experimental.pallas import tpu as pltpu

N_DEV = 8
M = 1536
N = 1536
P_PARTS = 3
PART_ROWS = M // P_PARTS
MASK_X, MASK_Y, MASK_Z = 1, 3, 4
ORDER = [
    [MASK_X, MASK_Y, MASK_Z],
    [MASK_Y, MASK_Z, MASK_X],
    [MASK_Z, MASK_X, MASK_Y],
]
EX = [PART_ROWS // 2, PART_ROWS // 4, PART_ROWS // 8]
SCR_OFF = [0, EX[0], EX[0] + EX[1]]
SCR_ROWS = EX[0] + EX[1] + EX[2]


def _side_bit(my, mask):
    if mask == MASK_X:
        return (my ^ (my >> 1)) & 1
    if mask == MASK_Y:
        return (my >> 1) & 1
    return (my >> 2) & 1


def kernel(A, B):
    def body(a_ref, b_ref, out_ref, p_ref, stage_ref, scr_ref, ag_ref,
             rs_send, rs_recv, ag_send, ag_recv):
        my = lax.axis_index("i")

        barrier = pltpu.get_barrier_semaphore()
        for mask in (MASK_X, MASK_Y, MASK_Z):
            pl.semaphore_signal(barrier, inc=1, device_id=(my ^ mask,),
                                device_id_type=pl.DeviceIdType.MESH)
        pl.semaphore_wait(barrier, 3)

        def mm(row0, rows):
            p_ref[pl.ds(row0, rows), :] = lax.dot_general(
                a_ref[pl.ds(row0, rows), :], b_ref[...],
                (((1,), (0,)), ((), ())),
                preferred_element_type=jnp.float32)

        def start_rs(part, s, send_lo):
            ex = EX[s]
            g_send = part * PART_ROWS + send_lo
            sb = part * SCR_ROWS + SCR_OFF[s]
            stage_ref[pl.ds(sb, ex), :] = (
                p_ref[pl.ds(g_send, ex), :].astype(jnp.bfloat16))
            rdma = pltpu.make_async_remote_copy(
                src_ref=stage_ref.at[pl.ds(sb, ex), :],
                dst_ref=scr_ref.at[pl.ds(sb, ex), :],
                send_sem=rs_send.at[part, s],
                recv_sem=rs_recv.at[part, s],
                device_id=(my ^ ORDER[part][s],),
                device_id_type=pl.DeviceIdType.MESH,
            )
            rdma.start()
            return rdma, sb

        lo = [0, 0, 0]
        started = []
        for part in range(P_PARTS):
            bbit = _side_bit(my, ORDER[part][0])
            send_lo = (1 - bbit) * EX[0]
            mm(part * PART_ROWS + send_lo, EX[0])
            started.append(start_rs(part, 0, send_lo))
            lo[part] = bbit * EX[0]
        for part in range(P_PARTS):
            mm(part * PART_ROWS + lo[part], EX[0])
        for part, (rdma, sb) in enumerate(started):
            rdma.wait()
            g = part * PART_ROWS + lo[part]
            p_ref[pl.ds(g, EX[0]), :] = (
                p_ref[pl.ds(g, EX[0]), :]
                + scr_ref[pl.ds(sb, EX[0]), :].astype(jnp.float32))

        for s in (1, 2):
            ex = EX[s]
            started = []
            for part in range(P_PARTS):
                bbit = _side_bit(my, ORDER[part][s])
                started.append(start_rs(part, s, lo[part] + (1 - bbit) * ex))
                lo[part] = lo[part] + bbit * ex
            for part, (rdma, sb) in enumerate(started):
                rdma.wait()
                g = part * PART_ROWS + lo[part]
                p_ref[pl.ds(g, ex), :] = (
                    p_ref[pl.ds(g, ex), :]
                    + scr_ref[pl.ds(sb, ex), :].astype(jnp.float32))

        for part in range(P_PARTS):
            g = part * PART_ROWS + lo[part]
            z = p_ref[pl.ds(g, EX[2]), :]
            act = z / (1.0 + jnp.exp(-z))
            out_ref[pl.ds(g, EX[2]), :] = act
            ag_ref[pl.ds(g, EX[2]), :] = act.astype(jnp.bfloat16)

        own = EX[2]
        for k, s in enumerate((2, 1, 0)):
            started = []
            for part in range(P_PARTS):
                bbit = _side_bit(my, ORDER[part][s])
                g = part * PART_ROWS + lo[part]
                rdma = pltpu.make_async_remote_copy(
                    src_ref=ag_ref.at[pl.ds(g, own), :],
                    dst_ref=ag_ref.at[pl.ds(g, own), :],
                    send_sem=ag_send.at[part, k],
                    recv_sem=ag_recv.at[part, k],
                    device_id=(my ^ ORDER[part][s],),
                    device_id_type=pl.DeviceIdType.MESH,
                )
                rdma.start()
                new_lo = lo[part] - bbit * own
                recv_lo = new_lo + (1 - bbit) * own
                started.append((rdma, part, recv_lo))
                lo[part] = new_lo
            for rdma, part, recv_lo in started:
                rdma.wait()
                g = part * PART_ROWS + recv_lo
                out_ref[pl.ds(g, own), :] = (
                    ag_ref[pl.ds(g, own), :].astype(jnp.float32))
            own *= 2

    return pl.pallas_call(
        body,
        out_shape=jax.ShapeDtypeStruct((M, N), jnp.float32),
        in_specs=[pl.BlockSpec(memory_space=pltpu.VMEM),
                  pl.BlockSpec(memory_space=pltpu.VMEM)],
        out_specs=pl.BlockSpec(memory_space=pltpu.VMEM),
        scratch_shapes=[
            pltpu.VMEM((M, N), jnp.float32),
            pltpu.VMEM((P_PARTS * SCR_ROWS, N), jnp.bfloat16),
            pltpu.VMEM((P_PARTS * SCR_ROWS, N), jnp.bfloat16),
            pltpu.VMEM((M, N), jnp.bfloat16),
            pltpu.SemaphoreType.DMA((P_PARTS, 3)),
            pltpu.SemaphoreType.DMA((P_PARTS, 3)),
            pltpu.SemaphoreType.DMA((P_PARTS, 3)),
            pltpu.SemaphoreType.DMA((P_PARTS, 3)),
        ],
        compiler_params=pltpu.CompilerParams(collective_id=0),
    )(A, B)


# device time: 54665 ns/iter; 4.0147x vs baseline; 1.0317x over previous
import jax
import jax.numpy as jnp
from jax import lax
from jax.experimental import pallas as pl
from jax.experimental.pallas import tpu as pltpu

N_DEV = 8
M = 1536
N = 1536
K = 768
P_PARTS = 3
PART_ROWS = M // P_PARTS
MASK_X, MASK_Y, MASK_Z = 1, 3, 4
ORDER = [
    [MASK_X, MASK_Y, MASK_Z],
    [MASK_Y, MASK_Z, MASK_X],
    [MASK_Z, MASK_X, MASK_Y],
]
EX = [PART_ROWS // 2, PART_ROWS // 4, PART_ROWS // 8]
SCR_OFF = [0, EX[0], EX[0] + EX[1]]
SCR_ROWS = EX[0] + EX[1] + EX[2]


def _side_bit(my, mask):
    if mask == MASK_X:
        return (my ^ (my >> 1)) & 1
    if mask == MASK_Y:
        return (my >> 1) & 1
    return (my >> 2) & 1


def kernel(A, B):
    def body(a_ref, b_ref, out_ref, abf_ref, bbf_ref, p_ref, stage_ref,
             scr_ref, ag_ref, rs_send, rs_recv, ag_send, ag_recv):
        my = lax.axis_index("i")

        barrier = pltpu.get_barrier_semaphore()
        for mask in (MASK_X, MASK_Y, MASK_Z):
            pl.semaphore_signal(barrier, inc=1, device_id=(my ^ mask,),
                                device_id_type=pl.DeviceIdType.MESH)

        abf_ref[...] = a_ref[...].astype(jnp.bfloat16)
        bbf_ref[...] = b_ref[...].astype(jnp.bfloat16)

        def mm(row0, rows):
            p_ref[pl.ds(row0, rows), :] = lax.dot_general(
                abf_ref[pl.ds(row0, rows), :], bbf_ref[...],
                (((1,), (0,)), ((), ())),
                preferred_element_type=jnp.float32)

        def make_rs(part, s, send_lo):
            ex = EX[s]
            sb = part * SCR_ROWS + SCR_OFF[s]
            stage_ref[pl.ds(sb, ex), :] = (
                p_ref[pl.ds(part * PART_ROWS + send_lo, ex), :]
                .astype(jnp.bfloat16))
            return pltpu.make_async_remote_copy(
                src_ref=stage_ref.at[pl.ds(sb, ex), :],
                dst_ref=scr_ref.at[pl.ds(sb, ex), :],
                send_sem=rs_send.at[part, s],
                recv_sem=rs_recv.at[part, s],
                device_id=(my ^ ORDER[part][s],),
                device_id_type=pl.DeviceIdType.MESH,
            ), sb

        lo = [0, 0, 0]
        rs = [[None] * 3 for _ in range(P_PARTS)]
        for part in range(P_PARTS):
            bbit = _side_bit(my, ORDER[part][0])
            send_lo = (1 - bbit) * EX[0]
            mm(part * PART_ROWS + send_lo, EX[0])
            rs[part][0] = make_rs(part, 0, send_lo)
            if part == 0:
                pl.semaphore_wait(barrier, 3)
            rs[part][0][0].start()
            lo[part] = bbit * EX[0]
        for part in range(P_PARTS):
            mm(part * PART_ROWS + lo[part], EX[0])

        for s in range(3):
            ex = EX[s]
            for part in range(P_PARTS):
                rdma, sb = rs[part][s]
                rdma.wait()
                g = part * PART_ROWS + lo[part]
                p_ref[pl.ds(g, ex), :] = (
                    p_ref[pl.ds(g, ex), :]
                    + scr_ref[pl.ds(sb, ex), :].astype(jnp.float32))
                if s < 2:
                    nbit = _side_bit(my, ORDER[part][s + 1])
                    nex = EX[s + 1]
                    rs[part][s + 1] = make_rs(
                        part, s + 1, lo[part] + (1 - nbit) * nex)
                    rs[part][s + 1][0].start()
                    lo[part] = lo[part] + nbit * nex

        def make_ag(part, k, g, own):
            return pltpu.make_async_remote_copy(
                src_ref=ag_ref.at[pl.ds(g, own), :],
                dst_ref=ag_ref.at[pl.ds(g, own), :],
                send_sem=ag_send.at[part, k],
                recv_sem=ag_recv.at[part, k],
                device_id=(my ^ ORDER[part][2 - k],),
                device_id_type=pl.DeviceIdType.MESH,
            )

        ag = [[None] * 3 for _ in range(P_PARTS)]
        for part in range(P_PARTS):
            g = part * PART_ROWS + lo[part]
            z = p_ref[pl.ds(g, EX[2]), :]
            act = z / (1.0 + jnp.exp(-z))
            out_ref[pl.ds(g, EX[2]), :] = act
            ag_ref[pl.ds(g, EX[2]), :] = act.astype(jnp.bfloat16)
            ag[part][0] = make_ag(part, 0, g, EX[2])
            ag[part][0].start()

        own = EX[2]
        for k in range(3):
            for part in range(P_PARTS):
                ag[part][k].wait()
                bbit = _side_bit(my, ORDER[part][2 - k])
                new_lo = lo[part] - bbit * own
                recv_lo = new_lo + (1 - bbit) * own
                lo[part] = new_lo
                if k < 2:
                    ag[part][k + 1] = make_ag(
                        part, k + 1, part * PART_ROWS + new_lo, 2 * own)
                    ag[part][k + 1].start()
                g = part * PART_ROWS + recv_lo
                out_ref[pl.ds(g, own), :] = (
                    ag_ref[pl.ds(g, own), :].astype(jnp.float32))
            own *= 2

    return pl.pallas_call(
        body,
        out_shape=jax.ShapeDtypeStruct((M, N), jnp.float32),
        in_specs=[pl.BlockSpec(memory_space=pltpu.VMEM),
                  pl.BlockSpec(memory_space=pltpu.VMEM)],
        out_specs=pl.BlockSpec(memory_space=pltpu.VMEM),
        scratch_shapes=[
            pltpu.VMEM((M, K), jnp.bfloat16),
            pltpu.VMEM((K, N), jnp.bfloat16),
            pltpu.VMEM((M, N), jnp.float32),
            pltpu.VMEM((P_PARTS * SCR_ROWS, N), jnp.bfloat16),
            pltpu.VMEM((P_PARTS * SCR_ROWS, N), jnp.bfloat16),
            pltpu.VMEM((M, N), jnp.bfloat16),
            pltpu.SemaphoreType.DMA((P_PARTS, 3)),
            pltpu.SemaphoreType.DMA((P_PARTS, 3)),
            pltpu.SemaphoreType.DMA((P_PARTS, 3)),
            pltpu.SemaphoreType.DMA((P_PARTS, 3)),
        ],
        compiler_params=pltpu.CompilerParams(collective_id=0),
    )(A, B)


# device time: 45276 ns/iter; 4.8472x vs baseline; 1.2074x over previous
import jax
import jax.numpy as jnp
from jax import lax
from jax.experimental import pallas as pl
from jax.experimental.pallas import tpu as pltpu

N_DEV = 8
M = 1536
N = 1536
K = 768
P_PARTS = 6
PART_ROWS = M // P_PARTS
MASK_X, MASK_Y, MASK_Z = 1, 3, 4
ORDER = [
    [MASK_X, MASK_Y, MASK_Z],
    [MASK_Y, MASK_Z, MASK_X],
    [MASK_Z, MASK_X, MASK_Y],
    [MASK_X, MASK_Y, MASK_Z],
    [MASK_Y, MASK_Z, MASK_X],
    [MASK_Z, MASK_X, MASK_Y],
]
EX = [PART_ROWS // 2, PART_ROWS // 4, PART_ROWS // 4]
SCR_OFF = [0, EX[0], EX[0] + EX[1]]
SCR_ROWS = EX[0] + EX[1] + EX[2]


def _side_bit(my, mask):
    if mask == MASK_X:
        return (my ^ (my >> 1)) & 1
    if mask == MASK_Y:
        return (my >> 1) & 1
    return (my >> 2) & 1


def kernel(A, B):
    def body(a_ref, b_ref, out_ref, abf_ref, bbf_ref, p_ref, stage_ref,
             scr_ref, ag_ref, rs_send, rs_recv, ag_send, ag_recv):
        my = lax.axis_index("i")

        barrier = pltpu.get_barrier_semaphore()
        for mask in (MASK_X, MASK_Y, MASK_Z):
            pl.semaphore_signal(barrier, inc=1, device_id=(my ^ mask,),
                                device_id_type=pl.DeviceIdType.MESH)

        abf_ref[...] = a_ref[...].astype(jnp.bfloat16)
        bbf_ref[...] = b_ref[...].astype(jnp.bfloat16)

        def mm(row0, rows):
            p_ref[pl.ds(row0, rows), :] = lax.dot_general(
                abf_ref[pl.ds(row0, rows), :], bbf_ref[...],
                (((1,), (0,)), ((), ())),
                preferred_element_type=jnp.float32)

        def make_rs(part, s, send_lo):
            ex = EX[s]
            sb = part * SCR_ROWS + SCR_OFF[s]
            stage_ref[pl.ds(sb, ex), :] = (
                p_ref[pl.ds(part * PART_ROWS + send_lo, ex), :]
                .astype(jnp.bfloat16))
            return pltpu.make_async_remote_copy(
                src_ref=stage_ref.at[pl.ds(sb, ex), :],
                dst_ref=scr_ref.at[pl.ds(sb, ex), :],
                send_sem=rs_send.at[part, s],
                recv_sem=rs_recv.at[part, s],
                device_id=(my ^ ORDER[part][s],),
                device_id_type=pl.DeviceIdType.MESH,
            ), sb

        def make_ag(part, k, g, rows):
            return pltpu.make_async_remote_copy(
                src_ref=ag_ref.at[pl.ds(g, rows), :],
                dst_ref=ag_ref.at[pl.ds(g, rows), :],
                send_sem=ag_send.at[part, k],
                recv_sem=ag_recv.at[part, k],
                device_id=(my ^ ORDER[part][1 - k],),
                device_id_type=pl.DeviceIdType.MESH,
            )

        lo = [0] * P_PARTS
        rs = [[None] * 3 for _ in range(P_PARTS)]
        for part in range(P_PARTS):
            bbit = _side_bit(my, ORDER[part][0])
            send_lo = (1 - bbit) * EX[0]
            mm(part * PART_ROWS + send_lo, EX[0])
            rs[part][0] = make_rs(part, 0, send_lo)
            if part == 0:
                pl.semaphore_wait(barrier, 3)
            rs[part][0][0].start()
            lo[part] = bbit * EX[0]
        for part in range(P_PARTS):
            mm(part * PART_ROWS + lo[part], EX[0])

        ag = [[None] * 2 for _ in range(P_PARTS)]
        for s in range(3):
            ex = EX[s]
            for part in range(P_PARTS):
                rdma, sb = rs[part][s]
                rdma.wait()
                g = part * PART_ROWS + lo[part]
                p_ref[pl.ds(g, ex), :] = (
                    p_ref[pl.ds(g, ex), :]
                    + scr_ref[pl.ds(sb, ex), :].astype(jnp.float32))
                if s == 0:
                    nbit = _side_bit(my, ORDER[part][1])
                    rs[part][1] = make_rs(
                        part, 1, lo[part] + (1 - nbit) * EX[1])
                    rs[part][1][0].start()
                    lo[part] = lo[part] + nbit * EX[1]
                elif s == 1:
                    rs[part][2] = make_rs(part, 2, lo[part])
                    rs[part][2][0].start()
                else:
                    z = p_ref[pl.ds(g, ex), :]
                    act = z / (1.0 + jnp.exp(-z))
                    out_ref[pl.ds(g, ex), :] = act
                    ag_ref[pl.ds(g, ex), :] = act.astype(jnp.bfloat16)
                    ag[part][0] = make_ag(part, 0, g, ex)
                    ag[part][0].start()

        own = EX[2]
        for k in range(2):
            for part in range(P_PARTS):
                ag[part][k].wait()
                bbit = _side_bit(my, ORDER[part][1 - k])
                new_lo = lo[part] - bbit * own
                recv_lo = new_lo + (1 - bbit) * own
                lo[part] = new_lo
                if k == 0:
                    ag[part][1] = make_ag(
                        part, 1, part * PART_ROWS + new_lo, 2 * own)
                    ag[part][1].start()
                g = part * PART_ROWS + recv_lo
                out_ref[pl.ds(g, own), :] = (
                    ag_ref[pl.ds(g, own), :].astype(jnp.float32))
            own *= 2

    return pl.pallas_call(
        body,
        out_shape=jax.ShapeDtypeStruct((M, N), jnp.float32),
        in_specs=[pl.BlockSpec(memory_space=pltpu.VMEM),
                  pl.BlockSpec(memory_space=pltpu.VMEM)],
        out_specs=pl.BlockSpec(memory_space=pltpu.VMEM),
        scratch_shapes=[
            pltpu.VMEM((M, K), jnp.bfloat16),
            pltpu.VMEM((K, N), jnp.bfloat16),
            pltpu.VMEM((M, N), jnp.float32),
            pltpu.VMEM((P_PARTS * SCR_ROWS, N), jnp.bfloat16),
            pltpu.VMEM((P_PARTS * SCR_ROWS, N), jnp.bfloat16),
            pltpu.VMEM((M, N), jnp.bfloat16),
            pltpu.SemaphoreType.DMA((P_PARTS, 3)),
            pltpu.SemaphoreType.DMA((P_PARTS, 3)),
            pltpu.SemaphoreType.DMA((P_PARTS, 2)),
            pltpu.SemaphoreType.DMA((P_PARTS, 2)),
        ],
        compiler_params=pltpu.CompilerParams(collective_id=0),
    )(A, B)
